# baseline (speedup 1.0000x reference)
import functools

import jax
import jax.numpy as jnp
from jax import lax
from jax.experimental import pallas as pl
from jax.experimental.pallas import tpu as pltpu
from jax.experimental.pallas import tpu_sc as plsc

BATCH = 16384
EMBED = 32
MLP_DIM = 128
CHUNK = 128


def _transpose_body(in_ref, out_ref):
    r = jax.lax.broadcasted_iota(jnp.int32, (EMBED, EMBED), 0)
    c = jax.lax.broadcasted_iota(jnp.int32, (EMBED, EMBED), 1)
    eye = (r == c).astype(jnp.float32)
    out_ref[...] = jax.lax.dot_general(
        in_ref[...], eye, (((0,), (0,)), ((), ())),
        preferred_element_type=jnp.float32)


def _untranspose(table_t, blkc=32768):
    v = table_t.shape[1]
    grid = (pl.cdiv(v, blkc),)
    return pl.pallas_call(
        _transpose_body,
        grid=grid,
        in_specs=[pl.BlockSpec((EMBED, blkc), lambda i: (0, i))],
        out_specs=pl.BlockSpec((blkc, EMBED), lambda i: (i, 0)),
        out_shape=jax.ShapeDtypeStruct((v, EMBED), jnp.float32),
    )(table_t)


def _sc_mlp_body(user_hbm, item_hbm, mlp_u_w, mlp_i_w,
                 out_mlp_u, out_mlp_i,
                 idx_u, idx_i, mlp_u_buf, mlp_i_buf, sem):
    info = plsc.get_sparse_core_info()
    nw = info.num_cores * info.num_subcores
    wid = lax.axis_index("s") * info.num_cores + lax.axis_index("c")
    b_per_w = BATCH // nw
    base = wid * b_per_w
    n_chunks = b_per_w // CHUNK
    for c in range(n_chunks):
        off = base + c * CHUNK
        pltpu.sync_copy(user_hbm.at[pl.ds(off, CHUNK)], idx_u)
        pltpu.sync_copy(item_hbm.at[pl.ds(off, CHUNK)], idx_i)
        c1 = pltpu.async_copy(mlp_u_w.at[idx_u], mlp_u_buf, sem)
        c2 = pltpu.async_copy(mlp_i_w.at[idx_i], mlp_i_buf, sem)
        c1.wait()
        c2.wait()
        pltpu.sync_copy(mlp_u_buf, out_mlp_u.at[pl.ds(off, CHUNK)])
        pltpu.sync_copy(mlp_i_buf, out_mlp_i.at[pl.ds(off, CHUNK)])


def _sc_gmf_body(user_hbm, item_hbm, gmf_u_w, gmf_i_w,
                 out_gmf_u, out_gmf_i,
                 idx_u, idx_i, gmf_u_buf, gmf_i_buf, sem2):
    info = plsc.get_sparse_core_info()
    nw = info.num_cores * info.num_subcores
    wid = lax.axis_index("s") * info.num_cores + lax.axis_index("c")
    b_per_w = BATCH // nw
    base = wid * b_per_w
    n_chunks = b_per_w // CHUNK
    for c in range(n_chunks):
        off = base + c * CHUNK
        pltpu.sync_copy(user_hbm.at[pl.ds(off, CHUNK)], idx_u)
        pltpu.sync_copy(item_hbm.at[pl.ds(off, CHUNK)], idx_i)

        def row_fetch(g, carry):
            vu = idx_u[pl.ds(g * 16, 16)]
            vi = idx_i[pl.ds(g * 16, 16)]
            for l in range(16):
                j = g * 16 + l
                pltpu.async_copy(gmf_u_w.at[pl.ds(vu[l], 1)],
                                 gmf_u_buf.at[pl.ds(j, 1)], sem2)
                pltpu.async_copy(gmf_i_w.at[pl.ds(vi[l], 1)],
                                 gmf_i_buf.at[pl.ds(j, 1)], sem2)
            return carry

        lax.fori_loop(0, CHUNK // 16, row_fetch, 0)
        pltpu.make_async_copy(gmf_u_w.at[pl.ds(0, CHUNK)], gmf_u_buf, sem2).wait()
        pltpu.make_async_copy(gmf_i_w.at[pl.ds(0, CHUNK)], gmf_i_buf, sem2).wait()
        pltpu.sync_copy(gmf_u_buf, out_gmf_u.at[pl.ds(off, CHUNK)])
        pltpu.sync_copy(gmf_i_buf, out_gmf_i.at[pl.ds(off, CHUNK)])


def _sc_mlp_gather(user, item, mlp_u_w, mlp_i_w):
    mesh = plsc.VectorSubcoreMesh(core_axis_name="c", subcore_axis_name="s")
    f = pl.kernel(
        _sc_mlp_body,
        mesh=mesh,
        compiler_params=pltpu.CompilerParams(use_tc_tiling_on_sc=True),
        out_type=(
            jax.ShapeDtypeStruct((BATCH, MLP_DIM), jnp.float32),
            jax.ShapeDtypeStruct((BATCH, MLP_DIM), jnp.float32),
        ),
        scratch_types=[
            pltpu.VMEM((CHUNK,), jnp.int32),
            pltpu.VMEM((CHUNK,), jnp.int32),
            pltpu.VMEM((CHUNK, MLP_DIM), jnp.float32),
            pltpu.VMEM((CHUNK, MLP_DIM), jnp.float32),
            pltpu.SemaphoreType.DMA,
        ],
    )
    return f(user, item, mlp_u_w, mlp_i_w)


def _sc_gmf_gather(user, item, gmf_u_w, gmf_i_w):
    mesh = plsc.VectorSubcoreMesh(core_axis_name="c", subcore_axis_name="s")
    f = pl.kernel(
        _sc_gmf_body,
        mesh=mesh,
        compiler_params=pltpu.CompilerParams(use_tc_tiling_on_sc=True),
        out_type=(
            jax.ShapeDtypeStruct((BATCH, EMBED), jnp.float32),
            jax.ShapeDtypeStruct((BATCH, EMBED), jnp.float32),
        ),
        scratch_types=[
            pltpu.VMEM((CHUNK,), jnp.int32),
            pltpu.VMEM((CHUNK,), jnp.int32),
            pltpu.VMEM((CHUNK, EMBED), jnp.float32),
            pltpu.VMEM((CHUNK, EMBED), jnp.float32),
            pltpu.SemaphoreType.DMA,
        ],
    )
    return f(user, item, gmf_u_w, gmf_i_w)


def _tc_dense_body(mlp_u_ref, mlp_i_ref, gmf_u_ref, gmf_i_ref,
                   w0u_ref, w0i_ref, b0_ref, w1_ref, b1_ref, w2_ref, b2_ref,
                   wpg_ref, wpx_ref, bp_ref, out_ref):
    h = jnp.dot(mlp_u_ref[...], w0u_ref[...], preferred_element_type=jnp.float32)
    h += jnp.dot(mlp_i_ref[...], w0i_ref[...], preferred_element_type=jnp.float32)
    h = jnp.maximum(h + b0_ref[...], 0.0)
    h = jnp.dot(h, w1_ref[...], preferred_element_type=jnp.float32)
    h = jnp.maximum(h + b1_ref[...], 0.0)
    h = jnp.dot(h, w2_ref[...], preferred_element_type=jnp.float32)
    h = jnp.maximum(h + b2_ref[...], 0.0)
    g = gmf_u_ref[...] * gmf_i_ref[...]
    p = jnp.sum(g * wpg_ref[...], axis=1, keepdims=True)
    p += jnp.sum(h * wpx_ref[...], axis=1, keepdims=True)
    out_ref[...] = p + bp_ref[...]


def _tc_dense(mlp_u, mlp_i, gmf_u, gmf_i, W0, b0, W1, b1, W2, b2, Wp, bp):
    blk = 2048
    grid = (BATCH // blk,)
    w0u_t = W0[:, :MLP_DIM].T
    w0i_t = W0[:, MLP_DIM:].T
    w1_t = W1.T
    w2_t = W2.T
    wpg = Wp[:, :EMBED]
    wpx = Wp[:, EMBED:]
    full = lambda s: pl.BlockSpec(s, lambda i: (0, 0))
    row = lambda w: pl.BlockSpec((blk, w), lambda i: (i, 0))
    out = pl.pallas_call(
        _tc_dense_body,
        grid=grid,
        in_specs=[
            row(MLP_DIM), row(MLP_DIM), row(EMBED), row(EMBED),
            full(w0u_t.shape), full(w0i_t.shape), full((1, 128)),
            full(w1_t.shape), full((1, 64)),
            full(w2_t.shape), full((1, 32)),
            full(wpg.shape), full(wpx.shape), full((1, 1)),
        ],
        out_specs=pl.BlockSpec((blk, 1), lambda i: (i, 0)),
        out_shape=jax.ShapeDtypeStruct((BATCH, 1), jnp.float32),
    )(mlp_u, mlp_i, gmf_u, gmf_i,
      w0u_t, w0i_t, b0.reshape(1, -1), w1_t, b1.reshape(1, -1),
      w2_t, b2.reshape(1, -1), wpg, wpx, bp.reshape(1, 1))
    return out.reshape(-1)


def kernel(user, item, gmf_user_w, gmf_item_w, mlp_user_w, mlp_item_w,
           W0, b0, W1, b1, W2, b2, Wp, bp):
    user = user.astype(jnp.int32)
    item = item.astype(jnp.int32)
    mlp_u, mlp_i = _sc_mlp_gather(user, item, mlp_user_w, mlp_item_w)
    gmf_u_fixed = _untranspose(gmf_user_w.T)
    gmf_i_fixed = _untranspose(gmf_item_w.T)
    gmf_u, gmf_i = _sc_gmf_gather(user, item, gmf_u_fixed, gmf_i_fixed)
    return _tc_dense(mlp_u, mlp_i, gmf_u, gmf_i,
                     W0, b0, W1, b1, W2, b2, Wp, bp)


# speedup vs baseline: 1.0363x; 1.0363x over previous
import functools

import jax
import jax.numpy as jnp
from jax import lax
from jax.experimental import pallas as pl
from jax.experimental.pallas import tpu as pltpu
from jax.experimental.pallas import tpu_sc as plsc

BATCH = 16384
EMBED = 32
MLP_DIM = 128
CHUNK = 128


def _transpose_body(in_ref, out_ref):
    r = jax.lax.broadcasted_iota(jnp.int32, (EMBED, EMBED), 0)
    c = jax.lax.broadcasted_iota(jnp.int32, (EMBED, EMBED), 1)
    eye = (r == c).astype(jnp.float32)
    out_ref[...] = jax.lax.dot_general(
        in_ref[...], eye, (((0,), (0,)), ((), ())),
        preferred_element_type=jnp.float32)


def _untranspose(table_t, blkc=40960):
    v = table_t.shape[1]
    grid = (pl.cdiv(v, blkc),)
    return pl.pallas_call(
        _transpose_body,
        grid=grid,
        in_specs=[pl.BlockSpec((EMBED, blkc), lambda i: (0, i))],
        out_specs=pl.BlockSpec((blkc, EMBED), lambda i: (i, 0)),
        out_shape=jax.ShapeDtypeStruct((v, EMBED), jnp.float32),
    )(table_t)


def _sc_gather_body(user_hbm, item_hbm, gmf_u_w, gmf_i_w, mlp_u_w, mlp_i_w,
                    out_mlp_u, out_mlp_i, out_gmf_u, out_gmf_i,
                    idx_u, idx_i,
                    mlp_u_buf, mlp_i_buf, gmf_u_buf, gmf_i_buf, sem, sem2):
    info = plsc.get_sparse_core_info()
    nw = info.num_cores * info.num_subcores
    wid = lax.axis_index("s") * info.num_cores + lax.axis_index("c")
    b_per_w = BATCH // nw
    base = wid * b_per_w
    n_chunks = b_per_w // CHUNK
    for c in range(n_chunks):
        off = base + c * CHUNK
        pltpu.sync_copy(user_hbm.at[pl.ds(off, CHUNK)], idx_u)
        pltpu.sync_copy(item_hbm.at[pl.ds(off, CHUNK)], idx_i)
        c1 = pltpu.async_copy(mlp_u_w.at[idx_u], mlp_u_buf, sem)
        c2 = pltpu.async_copy(mlp_i_w.at[idx_i], mlp_i_buf, sem)

        def row_fetch(g, carry):
            vu = idx_u[pl.ds(g * 16, 16)]
            vi = idx_i[pl.ds(g * 16, 16)]
            for l in range(16):
                j = g * 16 + l
                pltpu.async_copy(gmf_u_w.at[pl.ds(vu[l], 1)],
                                 gmf_u_buf.at[pl.ds(j, 1)], sem2)
                pltpu.async_copy(gmf_i_w.at[pl.ds(vi[l], 1)],
                                 gmf_i_buf.at[pl.ds(j, 1)], sem2)
            return carry

        lax.fori_loop(0, CHUNK // 16, row_fetch, 0)
        c1.wait()
        c2.wait()
        pltpu.make_async_copy(gmf_u_w.at[pl.ds(0, CHUNK)], gmf_u_buf, sem2).wait()
        pltpu.make_async_copy(gmf_i_w.at[pl.ds(0, CHUNK)], gmf_i_buf, sem2).wait()
        pltpu.sync_copy(mlp_u_buf, out_mlp_u.at[pl.ds(off, CHUNK)])
        pltpu.sync_copy(mlp_i_buf, out_mlp_i.at[pl.ds(off, CHUNK)])
        pltpu.sync_copy(gmf_u_buf, out_gmf_u.at[pl.ds(off, CHUNK)])
        pltpu.sync_copy(gmf_i_buf, out_gmf_i.at[pl.ds(off, CHUNK)])


def _sc_gather(user, item, gmf_u_w, gmf_i_w, mlp_u_w, mlp_i_w):
    mesh = plsc.VectorSubcoreMesh(core_axis_name="c", subcore_axis_name="s")
    f = pl.kernel(
        _sc_gather_body,
        mesh=mesh,
        compiler_params=pltpu.CompilerParams(use_tc_tiling_on_sc=True),
        out_type=(
            jax.ShapeDtypeStruct((BATCH, MLP_DIM), jnp.float32),
            jax.ShapeDtypeStruct((BATCH, MLP_DIM), jnp.float32),
            jax.ShapeDtypeStruct((BATCH, EMBED), jnp.float32),
            jax.ShapeDtypeStruct((BATCH, EMBED), jnp.float32),
        ),
        scratch_types=[
            pltpu.VMEM((CHUNK,), jnp.int32),
            pltpu.VMEM((CHUNK,), jnp.int32),
            pltpu.VMEM((CHUNK, MLP_DIM), jnp.float32),
            pltpu.VMEM((CHUNK, MLP_DIM), jnp.float32),
            pltpu.VMEM((CHUNK, EMBED), jnp.float32),
            pltpu.VMEM((CHUNK, EMBED), jnp.float32),
            pltpu.SemaphoreType.DMA,
            pltpu.SemaphoreType.DMA,
        ],
    )
    return f(user, item, gmf_u_w, gmf_i_w, mlp_u_w, mlp_i_w)


def _tc_dense_body(mlp_u_ref, mlp_i_ref, gmf_u_ref, gmf_i_ref,
                   w0u_ref, w0i_ref, b0_ref, w1_ref, b1_ref, w2_ref, b2_ref,
                   wpg_ref, wpx_ref, bp_ref, out_ref):
    h = jnp.dot(mlp_u_ref[...], w0u_ref[...], preferred_element_type=jnp.float32)
    h += jnp.dot(mlp_i_ref[...], w0i_ref[...], preferred_element_type=jnp.float32)
    h = jnp.maximum(h + b0_ref[...], 0.0)
    h = jnp.dot(h, w1_ref[...], preferred_element_type=jnp.float32)
    h = jnp.maximum(h + b1_ref[...], 0.0)
    h = jnp.dot(h, w2_ref[...], preferred_element_type=jnp.float32)
    h = jnp.maximum(h + b2_ref[...], 0.0)
    g = gmf_u_ref[...] * gmf_i_ref[...]
    p = jnp.sum(g * wpg_ref[...], axis=1, keepdims=True)
    p += jnp.sum(h * wpx_ref[...], axis=1, keepdims=True)
    out_ref[...] = p + bp_ref[...]


def _tc_dense(mlp_u, mlp_i, gmf_u, gmf_i, W0, b0, W1, b1, W2, b2, Wp, bp):
    blk = 2048
    grid = (BATCH // blk,)
    w0u_t = W0[:, :MLP_DIM].T
    w0i_t = W0[:, MLP_DIM:].T
    w1_t = W1.T
    w2_t = W2.T
    wpg = Wp[:, :EMBED]
    wpx = Wp[:, EMBED:]
    full = lambda s: pl.BlockSpec(s, lambda i: (0, 0))
    row = lambda w: pl.BlockSpec((blk, w), lambda i: (i, 0))
    out = pl.pallas_call(
        _tc_dense_body,
        grid=grid,
        in_specs=[
            row(MLP_DIM), row(MLP_DIM), row(EMBED), row(EMBED),
            full(w0u_t.shape), full(w0i_t.shape), full((1, 128)),
            full(w1_t.shape), full((1, 64)),
            full(w2_t.shape), full((1, 32)),
            full(wpg.shape), full(wpx.shape), full((1, 1)),
        ],
        out_specs=pl.BlockSpec((blk, 1), lambda i: (i, 0)),
        out_shape=jax.ShapeDtypeStruct((BATCH, 1), jnp.float32),
    )(mlp_u, mlp_i, gmf_u, gmf_i,
      w0u_t, w0i_t, b0.reshape(1, -1), w1_t, b1.reshape(1, -1),
      w2_t, b2.reshape(1, -1), wpg, wpx, bp.reshape(1, 1))
    return out.reshape(-1)


def kernel(user, item, gmf_user_w, gmf_item_w, mlp_user_w, mlp_item_w,
           W0, b0, W1, b1, W2, b2, Wp, bp):
    user = user.astype(jnp.int32)
    item = item.astype(jnp.int32)
    gmf_u_fixed = _untranspose(gmf_user_w.T)
    gmf_i_fixed = _untranspose(gmf_item_w.T)
    mlp_u, mlp_i, gmf_u, gmf_i = _sc_gather(
        user, item, gmf_u_fixed, gmf_i_fixed, mlp_user_w, mlp_item_w)
    return _tc_dense(mlp_u, mlp_i, gmf_u, gmf_i,
                     W0, b0, W1, b1, W2, b2, Wp, bp)
